# baseline (speedup 1.0000x reference)
import jax
import jax.numpy as jnp
from jax import lax
from jax.experimental import pallas as pl
from jax.experimental.pallas import tpu as pltpu
from jax.experimental.pallas import tpu_sc as plsc

N = 10000
F = 256
NC, NSUB, L = 2, 16, 16

WROWS = 320
NPAD = NC * NSUB * WROWS
W = 1024
CAP = W + 80

FP32 = jnp.float32
I32 = jnp.int32


def _pad_edges(e_pad, src, dst):
    e = src.shape[0]
    src = jnp.pad(src.astype(I32), (0, e_pad - e))
    dst = jnp.pad(dst.astype(I32), (0, e_pad - e), constant_values=-1)
    return src, dst


def _sc_mesh():
    return plsc.VectorSubcoreMesh(
        core_axis_name="c", subcore_axis_name="s",
        num_cores=NC, num_subcores=NSUB)


def _lane_sum(x):
    for sh in (8, 4, 2, 1):
        idx = lax.iota(I32, L) ^ sh
        x = x + jnp.take(x, idx)
    return x


def _incl_cumsum_i32(x):
    iot = lax.iota(I32, L)
    for sh in (1, 2, 4, 8):
        shifted = jnp.take(x, jnp.maximum(iot - sh, 0))
        x = x + jnp.where(iot >= sh, shifted, 0)
    return x


def _compact_window(off, wbase, dscan_v, sscan_v, cdst_v, csrc_v, ceid_v):
    iot = lax.iota(I32, L)

    def cg(j, cnt):
        d = dscan_v[pl.ds(j * L, L)]
        m = (d >= wbase) & (d < wbase + WROWS)
        sv = None if sscan_v is None else sscan_v[pl.ds(j * L, L)]

        mi = jnp.where(m, 1, 0)
        pc = _incl_cumsum_i32(mi)
        pcnt = pc[L - 1]

        def do(cnt):
            posm = jnp.where(m, pc - mi, L + iot)
            cd = jnp.where(iot < jnp.full((L,), pcnt), 0, wbase)
            cs = jnp.zeros((L,), I32)
            ce = jnp.zeros((L,), I32)
            for l in range(L):
                lv = jnp.full((L,), l, I32)
                eq = iot == jnp.take(posm, lv)
                cd = jnp.where(eq, jnp.take(d, lv), cd)
                if sscan_v is not None:
                    cs = jnp.where(eq, jnp.take(sv, lv), cs)
                if ceid_v is not None:
                    ce = jnp.where(eq, jnp.full((L,), off + j * L + l), ce)
            cdst_v[pl.ds(cnt, L)] = cd
            if sscan_v is not None:
                csrc_v[pl.ds(cnt, L)] = cs
            if ceid_v is not None:
                ceid_v[pl.ds(cnt, L)] = ce
            return cnt + pcnt
        return lax.cond(pcnt > 0, do, lambda cc: cc, cnt)
    cnt = lax.fori_loop(0, W // L, cg, jnp.int32(0))
    for g in range(4):
        cdst_v[pl.ds(cnt + g * L, L)] = jnp.full((L,), wbase, I32)
        if sscan_v is not None:
            csrc_v[pl.ds(cnt + g * L, L)] = jnp.zeros((L,), I32)
        if ceid_v is not None:
            ceid_v[pl.ds(cnt + g * L, L)] = jnp.zeros((L,), I32)
    return cnt


def _make_edge_call(e_pad, k, has_ew):
    scratch = [
        pltpu.VMEM((F,), FP32),
        pltpu.VMEM((F,), FP32),
        pltpu.VMEM((W,), I32),
        pltpu.VMEM((W,), I32),
        pltpu.VMEM((CAP,), I32),
        pltpu.VMEM((CAP,), I32),
    ] + ([pltpu.VMEM((CAP,), I32)] if has_ew else []) + [
        pltpu.VMEM((k, F), FP32),
        pltpu.VMEM((k, F), FP32),
    ] + ([pltpu.VMEM((k, F), FP32)] if has_ew else []) + [
        pltpu.VMEM((WROWS, F), FP32),
        pltpu.VMEM((WROWS + L,), FP32),
        pltpu.SemaphoreType.DMA,
        pltpu.SemaphoreType.DMA,
        pltpu.SemaphoreType.DMA,
    ]

    def body(src_h, dst_h, xl_h, xr_h, *rest):
        if has_ew:
            (ew_h, att_h, bias_h, ns_h, dsf_h, out_h,
             att_v, bias_v, dscan_v, sscan_v, cdst_v, csrc_v, ceid_v,
             xl_v, xr_v, ew_v, num_t, den_t, sem0, sem1, sem2) = rest
        else:
            (att_h, bias_h, ns_h, dsf_h, out_h,
             att_v, bias_v, dscan_v, sscan_v, cdst_v, csrc_v,
             xl_v, xr_v, num_t, den_t, sem0, sem1, sem2) = rest
            ew_h = ew_v = ceid_v = None
        c = lax.axis_index("c")
        s = lax.axis_index("s")
        wbase = (c * NSUB + s) * WROWS
        iot = lax.iota(I32, L)

        pltpu.sync_copy(att_h, att_v)
        pltpu.sync_copy(bias_h, bias_v)
        pltpu.sync_copy(ns_h.at[pl.ds(wbase, WROWS)], num_t)
        pltpu.sync_copy(dsf_h.at[pl.ds(wbase, WROWS + L)], den_t)

        def win(w, _):
            off = w * W
            pltpu.sync_copy(dst_h.at[pl.ds(off, W)], dscan_v)
            pltpu.sync_copy(src_h.at[pl.ds(off, W)], sscan_v)
            cnt = _compact_window(off, wbase, dscan_v, sscan_v,
                                  cdst_v, csrc_v, ceid_v)
            nblk = (cnt + (k - 1)) // k

            def pb(bi, _):
                done = bi * k
                cp0 = pltpu.async_copy(
                    xl_h.at[csrc_v.at[pl.ds(done, k)]], xl_v, sem0)
                cp1 = pltpu.async_copy(
                    xr_h.at[cdst_v.at[pl.ds(done, k)]], xr_v, sem1)
                if has_ew:
                    cp2 = pltpu.async_copy(
                        ew_h.at[ceid_v.at[pl.ds(done, k)]], ew_v, sem2)
                cp0.wait()
                cp1.wait()
                if has_ew:
                    cp2.wait()

                def grp(j, _):
                    dl = cdst_v[pl.ds(done + j * L, L)] - wbase
                    for r in range(L):
                        row = j * L + r
                        acc = jnp.zeros((L,), FP32)
                        for ci in range(F // L):
                            o = ci * L
                            v = xl_v[row, pl.ds(o, L)] + xr_v[row, pl.ds(o, L)]
                            if has_ew:
                                v = v + ew_v[row, pl.ds(o, L)]
                            v = jnp.where(v >= 0.0, v, 0.2 * v)
                            acc = acc + v * att_v[pl.ds(o, L)]
                        valid = (done + j * L + r) < cnt
                        ar = jnp.where(valid, jnp.exp(_lane_sum(acc)), 0.0)
                        dr = dl[r]
                        for ci in range(F // L):
                            o = ci * L
                            num_t[dr, pl.ds(o, L)] = (
                                num_t[dr, pl.ds(o, L)]
                                + xl_v[row, pl.ds(o, L)] * ar)
                        den_t[pl.ds(dr, L)] = (
                            den_t[pl.ds(dr, L)]
                            + jnp.where(iot == 0, ar, 0.0))
                    return 0
                lax.fori_loop(0, k // L, grp, 0)
                return 0
            lax.fori_loop(0, nblk, pb, 0)
            return 0
        lax.fori_loop(0, e_pad // W, win, 0)

        def nr(row, _):
            dv = den_t[pl.ds(row, L)]
            inv = 1.0 / jnp.full((L,), dv[0])
            for ci in range(F // L):
                o = ci * L
                num_t[row, pl.ds(o, L)] = (
                    num_t[row, pl.ds(o, L)] * inv + bias_v[pl.ds(o, L)])
            return 0
        lax.fori_loop(0, WROWS, nr, 0)
        pltpu.sync_copy(num_t, out_h.at[pl.ds(wbase, WROWS)])

    return pl.kernel(
        body, out_type=jax.ShapeDtypeStruct((NPAD, F), FP32),
        mesh=_sc_mesh(), scratch_types=scratch)


def _make_loopea_call(e_pad, ed):
    kb = 512
    scratch = [
        pltpu.VMEM((kb,), I32),
        pltpu.VMEM((kb, ed), FP32),
        pltpu.VMEM((WROWS * ed,), FP32),
        pltpu.VMEM((WROWS + L,), FP32),
    ]

    def body(dst_h, ea_h, out_h, dscan_v, ea_v, lea_t, cnt_t):
        c = lax.axis_index("c")
        s = lax.axis_index("s")
        wbase = (c * NSUB + s) * WROWS
        iot = lax.iota(I32, L)

        def zr(i, _):
            lea_t[pl.ds(i * L, L)] = jnp.zeros((L,), FP32)
            return 0
        lax.fori_loop(0, WROWS * ed // L, zr, 0)

        def zc(i, _):
            cnt_t[pl.ds(i * L, L)] = jnp.zeros((L,), FP32)
            return 0
        lax.fori_loop(0, (WROWS + L) // L, zc, 0)

        def blk(b, _):
            off = b * kb
            pltpu.sync_copy(dst_h.at[pl.ds(off, kb)], dscan_v)
            pltpu.sync_copy(ea_h.at[pl.ds(off, kb)], ea_v)

            def grp(j, _):
                d = dscan_v[pl.ds(j * L, L)]
                m = (d >= wbase) & (d < wbase + WROWS)
                mi = jnp.where(m, 1, 0)
                any_m = _lane_sum(mi)[0]
                m01 = jnp.where(m, 1.0, 0.0)
                dl = jnp.where(m, d - wbase, 0)

                def do(_):
                    for r in range(L):
                        row = j * L + r
                        dr = dl[r]
                        mr = jnp.take(m01, jnp.full((L,), r, I32))
                        rv = ea_v[row, pl.ds(0, ed)] * mr
                        lea_t[pl.ds(dr * ed, ed)] = (
                            lea_t[pl.ds(dr * ed, ed)] + rv)
                        cnt_t[pl.ds(dr, L)] = (
                            cnt_t[pl.ds(dr, L)]
                            + jnp.where(iot == 0, mr, 0.0))
                    return 0
                lax.cond(any_m > 0, do, lambda z: z, 0)
                return 0
            lax.fori_loop(0, kb // L, grp, 0)
            return 0
        lax.fori_loop(0, e_pad // kb, blk, 0)

        def nr(row, _):
            cv = cnt_t[pl.ds(row, L)]
            inv = 1.0 / jnp.maximum(jnp.full((L,), cv[0]), 1.0)
            lea_t[pl.ds(row * ed, ed)] = lea_t[pl.ds(row * ed, ed)] * inv
            return 0
        lax.fori_loop(0, WROWS, nr, 0)
        pltpu.sync_copy(lea_t, out_h.at[pl.ds(wbase * ed, WROWS * ed)])

    return pl.kernel(
        body, out_type=jax.ShapeDtypeStruct((NPAD * ed,), FP32),
        mesh=_sc_mesh(), scratch_types=scratch)


def _leaky(x):
    return jnp.where(x >= 0, x, 0.2 * x)


def _bnleaky_kernel(x_ref, g_ref, b_ref, o_ref):
    y = _leaky(x_ref[...])
    mu = jnp.mean(y, axis=0)
    var = jnp.mean((y - mu) ** 2, axis=0)
    o_ref[...] = (y - mu) / jnp.sqrt(var + 1e-5) * g_ref[...] + b_ref[...]


def _bnleaky(x, gamma, beta):
    return pl.pallas_call(
        _bnleaky_kernel,
        out_shape=jax.ShapeDtypeStruct(x.shape, FP32),
    )(x, gamma, beta)


def _prep_seg_kernel(y_ref, wl_ref, wr_ref, att_ref, lea_ref, we_ref,
                     xl_o, xr_o, ns_o, ds_o):
    y = y_ref[...]
    xl = jnp.dot(y, wl_ref[...], preferred_element_type=FP32)
    xr = jnp.dot(y, wr_ref[...], preferred_element_type=FP32)
    sv = xl + xr + jnp.dot(lea_ref[...], we_ref[...], preferred_element_type=FP32)
    lg = jnp.dot(_leaky(sv), att_ref[...], preferred_element_type=FP32)
    a = jnp.exp(lg)
    xl_o[...] = xl
    xr_o[...] = xr
    ns_o[...] = a * xl
    ds_o[...] = a


def _prep_dense_kernel(y_ref, wl_ref, wr_ref, att_ref, xl_o, xr_o, ns_o, ds_o):
    y = y_ref[...]
    xl = jnp.dot(y, wl_ref[...], preferred_element_type=FP32)
    xr = jnp.dot(y, wr_ref[...], preferred_element_type=FP32)
    lg = jnp.dot(_leaky(xl + xr), att_ref[...], preferred_element_type=FP32)
    a = jnp.exp(lg)
    xl_o[...] = xl
    xr_o[...] = xr
    ns_o[...] = a * xl
    ds_o[...] = a


def _prep(y, wl, wr, att, lea=None, we=None):
    nb = 10
    bn = N // nb
    cin = y.shape[1]
    outs = [jax.ShapeDtypeStruct((N, F), FP32)] * 3 + [jax.ShapeDtypeStruct((N, 1), FP32)]
    row = lambda w: pl.BlockSpec((bn, w), lambda i: (i, 0))
    rep = lambda a, b: pl.BlockSpec((a, b), lambda i: (0, 0))
    out_specs = [row(F), row(F), row(F), row(1)]
    if lea is not None:
        xl, xr, ns, dsf = pl.pallas_call(
            _prep_seg_kernel, grid=(nb,),
            in_specs=[row(cin), rep(cin, F), rep(cin, F), rep(F, 1),
                      row(16), rep(16, F)],
            out_specs=out_specs, out_shape=outs,
        )(y, wl, wr, att.reshape(F, 1), lea, we)
    else:
        xl, xr, ns, dsf = pl.pallas_call(
            _prep_dense_kernel, grid=(nb,),
            in_specs=[row(cin), rep(cin, F), rep(cin, F), rep(F, 1)],
            out_specs=out_specs, out_shape=outs,
        )(y, wl, wr, att.reshape(F, 1))
    return xl, xr, ns, dsf


def _eaw_kernel(a_ref, w_ref, o_ref):
    o_ref[...] = jnp.dot(a_ref[...], w_ref[...], preferred_element_type=FP32)


def _eaw(ea_pad, we):
    e_pad = ea_pad.shape[0]
    blk = 1024
    return pl.pallas_call(
        _eaw_kernel, grid=(e_pad // blk,),
        in_specs=[pl.BlockSpec((blk, 16), lambda i: (i, 0)),
                  pl.BlockSpec((16, F), lambda i: (0, 0))],
        out_specs=pl.BlockSpec((blk, F), lambda i: (i, 0)),
        out_shape=jax.ShapeDtypeStruct((e_pad, F), FP32),
    )(ea_pad, we)


def _pool_kernel(y_ref, b_ref, o_ref):
    y = y_ref[...]
    bids = b_ref[...]
    io = lax.broadcasted_iota(I32, (32, N), 0)
    oh = (io == bids).astype(FP32)
    s = jnp.dot(oh, y, preferred_element_type=FP32)
    cnt = jnp.sum(oh, axis=1, keepdims=True)
    o_ref[...] = s / jnp.clip(cnt, 1.0, None)


def _pool(y, batch):
    return pl.pallas_call(
        _pool_kernel,
        out_shape=jax.ShapeDtypeStruct((32, F), FP32),
    )(y, batch.astype(I32).reshape(1, N))


def _global_kernel(x_ref, w_ref, b_ref, g_ref, be_ref, o_ref):
    h = _leaky(jnp.dot(x_ref[...], w_ref[...], preferred_element_type=FP32)
               + b_ref[...])
    mu = jnp.mean(h, axis=0)
    var = jnp.mean((h - mu) ** 2, axis=0)
    o_ref[...] = (h - mu) / jnp.sqrt(var + 1e-5) * g_ref[...] + be_ref[...]


def _head_kernel(g_ref, s_ref, d_ref, w_ref, b_ref, o_ref):
    cat = jnp.concatenate([g_ref[...], s_ref[...], d_ref[...]], axis=1)
    logits = jnp.dot(cat, w_ref[...], preferred_element_type=FP32) + b_ref[...]
    m = jnp.max(logits, axis=1, keepdims=True)
    e = jnp.exp(logits - m)
    o_ref[...] = e / jnp.sum(e, axis=1, keepdims=True)


def _pad_self(ns, dsf):
    ns_p = jnp.pad(ns, ((0, NPAD - N), (0, 0)))
    d_p = jnp.pad(dsf.reshape(N), (0, NPAD + L - N), constant_values=1.0)
    return ns_p, d_p


def _gat_path(x, src, dst, e_pad, params, prefixes, batch, ea=None):
    has_ew = ea is not None
    src_p, dst_p = _pad_edges(e_pad, src, dst)
    edge_call = _make_edge_call(e_pad, 48 if has_ew else 64, has_ew)
    if has_ew:
        ea_pad = jnp.pad(ea, ((0, e_pad - ea.shape[0]), (0, 0)))
        lea = _make_loopea_call(e_pad, 16)(dst_p, ea_pad).reshape(NPAD, 16)[:N]
    y = x
    for li, pre in enumerate(prefixes):
        if li > 0:
            prev = prefixes[li - 1]
            y = _bnleaky(y, params[prev + '_gamma'], params[prev + '_beta'])
        if has_ew:
            xl, xr, ns, dsf = _prep(y, params[pre + '_Wl'], params[pre + '_Wr'],
                                    params[pre + '_att'], lea, params[pre + '_We'])
            ns_p, ds_p = _pad_self(ns, dsf)
            eaw = _eaw(ea_pad, params[pre + '_We'])
            y = edge_call(src_p, dst_p, xl, xr, eaw,
                          params[pre + '_att'], params[pre + '_b'],
                          ns_p, ds_p)[:N]
        else:
            xl, xr, ns, dsf = _prep(y, params[pre + '_Wl'], params[pre + '_Wr'],
                                    params[pre + '_att'])
            ns_p, ds_p = _pad_self(ns, dsf)
            y = edge_call(src_p, dst_p, xl, xr,
                          params[pre + '_att'], params[pre + '_b'],
                          ns_p, ds_p)[:N]
    last = prefixes[-1]
    y = _bnleaky(y, params[last + '_gamma'], params[last + '_beta'])
    return _pool(y, batch)


def kernel(global_data, segment_x, segment_edge_index, segment_edge_attr,
           segment_batch, dense_x, dense_edge_index, dense_batch, params):
    p = params
    g = pl.pallas_call(
        _global_kernel,
        out_shape=jax.ShapeDtypeStruct((32, F), FP32),
    )(global_data, p['g_W'], p['g_b'], p['g_gamma'], p['g_beta'])

    s_pool = _gat_path(segment_x, segment_edge_index[0], segment_edge_index[1],
                       160768, p, ('s0', 's1'), segment_batch,
                       ea=segment_edge_attr)
    d_pool = _gat_path(dense_x, dense_edge_index[0], dense_edge_index[1],
                       320512, p, ('d0', 'd1'), dense_batch)

    return pl.pallas_call(
        _head_kernel,
        out_shape=jax.ShapeDtypeStruct((32, 2), FP32),
    )(g, s_pool, d_pool, p['o_W'], p['o_b'])


# speedup vs baseline: 1.4620x; 1.4620x over previous
import jax
import jax.numpy as jnp
from jax import lax
from jax.experimental import pallas as pl
from jax.experimental.pallas import tpu as pltpu
from jax.experimental.pallas import tpu_sc as plsc

N = 10000
F = 256
NC, NSUB, L = 2, 16, 16

WROWS = 320
NPAD = NC * NSUB * WROWS
W = 1024
WD = 2048
CAP = W + 80
CAPD = WD + 80

FP32 = jnp.float32
I32 = jnp.int32


def _pad_edges(e_pad, src, dst):
    e = src.shape[0]
    src = jnp.pad(src.astype(I32), (0, e_pad - e))
    dst = jnp.pad(dst.astype(I32), (0, e_pad - e), constant_values=-1)
    return src, dst


def _sc_mesh():
    return plsc.VectorSubcoreMesh(
        core_axis_name="c", subcore_axis_name="s",
        num_cores=NC, num_subcores=NSUB)


def _lane_sum(x):
    for sh in (8, 4, 2, 1):
        idx = lax.iota(I32, L) ^ sh
        x = x + jnp.take(x, idx)
    return x


def _incl_cumsum_i32(x):
    iot = lax.iota(I32, L)
    for sh in (1, 2, 4, 8):
        shifted = jnp.take(x, jnp.maximum(iot - sh, 0))
        x = x + jnp.where(iot >= sh, shifted, 0)
    return x


def _compact_window(off, wbase, dscan_v, sscan_v, cdst_v, csrc_v, ceid_v, w):
    iot = lax.iota(I32, L)

    def cg(j, cnt):
        d = dscan_v[pl.ds(j * L, L)]
        m = (d >= wbase) & (d < wbase + WROWS)
        sv = None if sscan_v is None else sscan_v[pl.ds(j * L, L)]

        mi = jnp.where(m, 1, 0)
        pc = _incl_cumsum_i32(mi)
        pcnt = pc[L - 1]

        def do(cnt):
            posm = jnp.where(m, pc - mi, L + iot)
            cd = jnp.where(iot < jnp.full((L,), pcnt), 0, wbase)
            cs = jnp.zeros((L,), I32)
            ce = jnp.zeros((L,), I32)
            for l in range(L):
                lv = jnp.full((L,), l, I32)
                eq = iot == jnp.take(posm, lv)
                cd = jnp.where(eq, jnp.take(d, lv), cd)
                if sscan_v is not None:
                    cs = jnp.where(eq, jnp.take(sv, lv), cs)
                if ceid_v is not None:
                    ce = jnp.where(eq, jnp.full((L,), off + j * L + l), ce)
            cdst_v[pl.ds(cnt, L)] = cd
            if sscan_v is not None:
                csrc_v[pl.ds(cnt, L)] = cs
            if ceid_v is not None:
                ceid_v[pl.ds(cnt, L)] = ce
            return cnt + pcnt
        return lax.cond(pcnt > 0, do, lambda cc: cc, cnt)
    cnt = lax.fori_loop(0, w // L, cg, jnp.int32(0))
    for g in range(4):
        cdst_v[pl.ds(cnt + g * L, L)] = jnp.full((L,), wbase, I32)
        if sscan_v is not None:
            csrc_v[pl.ds(cnt + g * L, L)] = jnp.zeros((L,), I32)
        if ceid_v is not None:
            ceid_v[pl.ds(cnt + g * L, L)] = jnp.zeros((L,), I32)
    return cnt


def _make_edge_call(e_pad, k, has_ew):
    w = W if has_ew else WD
    cap = CAP if has_ew else CAPD
    scratch = [
        pltpu.VMEM((F,), FP32),
        pltpu.VMEM((F,), FP32),
        pltpu.VMEM((w,), I32),
        pltpu.VMEM((w,), I32),
        pltpu.VMEM((cap,), I32),
        pltpu.VMEM((cap,), I32),
    ] + ([pltpu.VMEM((cap,), I32)] if has_ew else []) + [
        pltpu.VMEM((k, F), FP32),
        pltpu.VMEM((k, F), FP32),
    ] + ([pltpu.VMEM((k, F), FP32)] if has_ew else []) + [
        pltpu.VMEM((WROWS, F), FP32),
        pltpu.VMEM((WROWS + L,), FP32),
        pltpu.SemaphoreType.DMA,
        pltpu.SemaphoreType.DMA,
        pltpu.SemaphoreType.DMA,
    ]

    def body(src_h, dst_h, xl_h, xr_h, *rest):
        if has_ew:
            (ew_h, att_h, bias_h, ns_h, dsf_h, out_h,
             att_v, bias_v, dscan_v, sscan_v, cdst_v, csrc_v, ceid_v,
             xl_v, xr_v, ew_v, num_t, den_t, sem0, sem1, sem2) = rest
        else:
            (att_h, bias_h, ns_h, dsf_h, out_h,
             att_v, bias_v, dscan_v, sscan_v, cdst_v, csrc_v,
             xl_v, xr_v, num_t, den_t, sem0, sem1, sem2) = rest
            ew_h = ew_v = ceid_v = None
        c = lax.axis_index("c")
        s = lax.axis_index("s")
        wbase = (c * NSUB + s) * WROWS
        iot = lax.iota(I32, L)

        pltpu.sync_copy(att_h, att_v)
        pltpu.sync_copy(bias_h, bias_v)
        pltpu.sync_copy(ns_h.at[pl.ds(wbase, WROWS)], num_t)
        pltpu.sync_copy(dsf_h.at[pl.ds(wbase, WROWS + L)], den_t)

        def win(wi, _):
            off = wi * w
            pltpu.sync_copy(dst_h.at[pl.ds(off, w)], dscan_v)
            pltpu.sync_copy(src_h.at[pl.ds(off, w)], sscan_v)
            cnt = _compact_window(off, wbase, dscan_v, sscan_v,
                                  cdst_v, csrc_v, ceid_v, w)
            nblk = (cnt + (k - 1)) // k

            def pb(bi, _):
                done = bi * k
                cp0 = pltpu.async_copy(
                    xl_h.at[csrc_v.at[pl.ds(done, k)]], xl_v, sem0)
                cp1 = pltpu.async_copy(
                    xr_h.at[cdst_v.at[pl.ds(done, k)]], xr_v, sem1)
                if has_ew:
                    cp2 = pltpu.async_copy(
                        ew_h.at[ceid_v.at[pl.ds(done, k)]], ew_v, sem2)
                cp0.wait()
                cp1.wait()
                if has_ew:
                    cp2.wait()

                def grp(j, _):
                    dl = cdst_v[pl.ds(done + j * L, L)] - wbase
                    for r in range(L):
                        row = j * L + r
                        acc = jnp.zeros((L,), FP32)
                        for ci in range(F // L):
                            o = ci * L
                            v = xl_v[row, pl.ds(o, L)] + xr_v[row, pl.ds(o, L)]
                            if has_ew:
                                v = v + ew_v[row, pl.ds(o, L)]
                            v = jnp.where(v >= 0.0, v, 0.2 * v)
                            acc = acc + v * att_v[pl.ds(o, L)]
                        valid = (done + j * L + r) < cnt
                        ar = jnp.where(valid, jnp.exp(_lane_sum(acc)), 0.0)
                        dr = dl[r]
                        for ci in range(F // L):
                            o = ci * L
                            num_t[dr, pl.ds(o, L)] = (
                                num_t[dr, pl.ds(o, L)]
                                + xl_v[row, pl.ds(o, L)] * ar)
                        den_t[pl.ds(dr, L)] = (
                            den_t[pl.ds(dr, L)]
                            + jnp.where(iot == 0, ar, 0.0))
                    return 0
                lax.fori_loop(0, k // L, grp, 0)
                return 0
            lax.fori_loop(0, nblk, pb, 0)
            return 0
        lax.fori_loop(0, e_pad // w, win, 0)

        def nr(row, _):
            dv = den_t[pl.ds(row, L)]
            inv = 1.0 / jnp.full((L,), dv[0])
            for ci in range(F // L):
                o = ci * L
                num_t[row, pl.ds(o, L)] = (
                    num_t[row, pl.ds(o, L)] * inv + bias_v[pl.ds(o, L)])
            return 0
        lax.fori_loop(0, WROWS, nr, 0)
        pltpu.sync_copy(num_t, out_h.at[pl.ds(wbase, WROWS)])

    return pl.kernel(
        body, out_type=jax.ShapeDtypeStruct((NPAD, F), FP32),
        mesh=_sc_mesh(), scratch_types=scratch)


def _make_loopea_call(e_pad, ed):
    kb = 512
    scratch = [
        pltpu.VMEM((kb,), I32),
        pltpu.VMEM((kb, ed), FP32),
        pltpu.VMEM((WROWS * ed,), FP32),
        pltpu.VMEM((WROWS + L,), FP32),
    ]

    def body(dst_h, ea_h, out_h, dscan_v, ea_v, lea_t, cnt_t):
        c = lax.axis_index("c")
        s = lax.axis_index("s")
        wbase = (c * NSUB + s) * WROWS
        iot = lax.iota(I32, L)

        def zr(i, _):
            lea_t[pl.ds(i * L, L)] = jnp.zeros((L,), FP32)
            return 0
        lax.fori_loop(0, WROWS * ed // L, zr, 0)

        def zc(i, _):
            cnt_t[pl.ds(i * L, L)] = jnp.zeros((L,), FP32)
            return 0
        lax.fori_loop(0, (WROWS + L) // L, zc, 0)

        def blk(b, _):
            off = b * kb
            pltpu.sync_copy(dst_h.at[pl.ds(off, kb)], dscan_v)
            pltpu.sync_copy(ea_h.at[pl.ds(off, kb)], ea_v)

            def grp(j, _):
                d = dscan_v[pl.ds(j * L, L)]
                m = (d >= wbase) & (d < wbase + WROWS)
                mi = jnp.where(m, 1, 0)
                any_m = _lane_sum(mi)[0]
                m01 = jnp.where(m, 1.0, 0.0)
                dl = jnp.where(m, d - wbase, 0)

                def do(_):
                    for r in range(L):
                        row = j * L + r
                        dr = dl[r]
                        mr = jnp.take(m01, jnp.full((L,), r, I32))
                        rv = ea_v[row, pl.ds(0, ed)] * mr
                        lea_t[pl.ds(dr * ed, ed)] = (
                            lea_t[pl.ds(dr * ed, ed)] + rv)
                        cnt_t[pl.ds(dr, L)] = (
                            cnt_t[pl.ds(dr, L)]
                            + jnp.where(iot == 0, mr, 0.0))
                    return 0
                lax.cond(any_m > 0, do, lambda z: z, 0)
                return 0
            lax.fori_loop(0, kb // L, grp, 0)
            return 0
        lax.fori_loop(0, e_pad // kb, blk, 0)

        def nr(row, _):
            cv = cnt_t[pl.ds(row, L)]
            inv = 1.0 / jnp.maximum(jnp.full((L,), cv[0]), 1.0)
            lea_t[pl.ds(row * ed, ed)] = lea_t[pl.ds(row * ed, ed)] * inv
            return 0
        lax.fori_loop(0, WROWS, nr, 0)
        pltpu.sync_copy(lea_t, out_h.at[pl.ds(wbase * ed, WROWS * ed)])

    return pl.kernel(
        body, out_type=jax.ShapeDtypeStruct((NPAD * ed,), FP32),
        mesh=_sc_mesh(), scratch_types=scratch)


def _leaky(x):
    return jnp.where(x >= 0, x, 0.2 * x)


def _bnleaky_kernel(x_ref, g_ref, b_ref, o_ref):
    y = _leaky(x_ref[...])
    mu = jnp.mean(y, axis=0)
    var = jnp.mean((y - mu) ** 2, axis=0)
    o_ref[...] = (y - mu) / jnp.sqrt(var + 1e-5) * g_ref[...] + b_ref[...]


def _bnleaky(x, gamma, beta):
    return pl.pallas_call(
        _bnleaky_kernel,
        out_shape=jax.ShapeDtypeStruct(x.shape, FP32),
    )(x, gamma, beta)


def _prep_seg_kernel(y_ref, wl_ref, wr_ref, att_ref, lea_ref, we_ref,
                     xl_o, xr_o, ns_o, ds_o):
    y = y_ref[...]
    xl = jnp.dot(y, wl_ref[...], preferred_element_type=FP32)
    xr = jnp.dot(y, wr_ref[...], preferred_element_type=FP32)
    sv = xl + xr + jnp.dot(lea_ref[...], we_ref[...], preferred_element_type=FP32)
    lg = jnp.dot(_leaky(sv), att_ref[...], preferred_element_type=FP32)
    a = jnp.exp(lg)
    xl_o[...] = xl
    xr_o[...] = xr
    ns_o[...] = a * xl
    ds_o[...] = a


def _prep_dense_kernel(y_ref, wl_ref, wr_ref, att_ref, xl_o, xr_o, ns_o, ds_o):
    y = y_ref[...]
    xl = jnp.dot(y, wl_ref[...], preferred_element_type=FP32)
    xr = jnp.dot(y, wr_ref[...], preferred_element_type=FP32)
    lg = jnp.dot(_leaky(xl + xr), att_ref[...], preferred_element_type=FP32)
    a = jnp.exp(lg)
    xl_o[...] = xl
    xr_o[...] = xr
    ns_o[...] = a * xl
    ds_o[...] = a


def _prep(y, wl, wr, att, lea=None, we=None):
    nb = 10
    bn = N // nb
    cin = y.shape[1]
    outs = [jax.ShapeDtypeStruct((N, F), FP32)] * 3 + [jax.ShapeDtypeStruct((N, 1), FP32)]
    row = lambda w: pl.BlockSpec((bn, w), lambda i: (i, 0))
    rep = lambda a, b: pl.BlockSpec((a, b), lambda i: (0, 0))
    out_specs = [row(F), row(F), row(F), row(1)]
    if lea is not None:
        xl, xr, ns, dsf = pl.pallas_call(
            _prep_seg_kernel, grid=(nb,),
            in_specs=[row(cin), rep(cin, F), rep(cin, F), rep(F, 1),
                      row(16), rep(16, F)],
            out_specs=out_specs, out_shape=outs,
        )(y, wl, wr, att.reshape(F, 1), lea, we)
    else:
        xl, xr, ns, dsf = pl.pallas_call(
            _prep_dense_kernel, grid=(nb,),
            in_specs=[row(cin), rep(cin, F), rep(cin, F), rep(F, 1)],
            out_specs=out_specs, out_shape=outs,
        )(y, wl, wr, att.reshape(F, 1))
    return xl, xr, ns, dsf


def _eaw_kernel(a_ref, w_ref, o_ref):
    o_ref[...] = jnp.dot(a_ref[...], w_ref[...], preferred_element_type=FP32)


def _eaw(ea_pad, we):
    e_pad = ea_pad.shape[0]
    blk = 1024
    return pl.pallas_call(
        _eaw_kernel, grid=(e_pad // blk,),
        in_specs=[pl.BlockSpec((blk, 16), lambda i: (i, 0)),
                  pl.BlockSpec((16, F), lambda i: (0, 0))],
        out_specs=pl.BlockSpec((blk, F), lambda i: (i, 0)),
        out_shape=jax.ShapeDtypeStruct((e_pad, F), FP32),
    )(ea_pad, we)


def _pool_kernel(y_ref, b_ref, o_ref):
    y = y_ref[...]
    bids = b_ref[...]
    io = lax.broadcasted_iota(I32, (32, N), 0)
    oh = (io == bids).astype(FP32)
    s = jnp.dot(oh, y, preferred_element_type=FP32)
    cnt = jnp.sum(oh, axis=1, keepdims=True)
    o_ref[...] = s / jnp.clip(cnt, 1.0, None)


def _pool(y, batch):
    return pl.pallas_call(
        _pool_kernel,
        out_shape=jax.ShapeDtypeStruct((32, F), FP32),
    )(y, batch.astype(I32).reshape(1, N))


def _global_kernel(x_ref, w_ref, b_ref, g_ref, be_ref, o_ref):
    h = _leaky(jnp.dot(x_ref[...], w_ref[...], preferred_element_type=FP32)
               + b_ref[...])
    mu = jnp.mean(h, axis=0)
    var = jnp.mean((h - mu) ** 2, axis=0)
    o_ref[...] = (h - mu) / jnp.sqrt(var + 1e-5) * g_ref[...] + be_ref[...]


def _head_kernel(g_ref, s_ref, d_ref, w_ref, b_ref, o_ref):
    cat = jnp.concatenate([g_ref[...], s_ref[...], d_ref[...]], axis=1)
    logits = jnp.dot(cat, w_ref[...], preferred_element_type=FP32) + b_ref[...]
    m = jnp.max(logits, axis=1, keepdims=True)
    e = jnp.exp(logits - m)
    o_ref[...] = e / jnp.sum(e, axis=1, keepdims=True)


def _pad_self(ns, dsf):
    ns_p = jnp.pad(ns, ((0, NPAD - N), (0, 0)))
    d_p = jnp.pad(dsf.reshape(N), (0, NPAD + L - N), constant_values=1.0)
    return ns_p, d_p


def _gat_path(x, src, dst, e_pad, params, prefixes, batch, ea=None):
    has_ew = ea is not None
    src_p, dst_p = _pad_edges(e_pad, src, dst)
    edge_call = _make_edge_call(e_pad, 48 if has_ew else 64, has_ew)
    if has_ew:
        ea_pad = jnp.pad(ea, ((0, e_pad - ea.shape[0]), (0, 0)))
        lea = _make_loopea_call(e_pad, 16)(dst_p, ea_pad).reshape(NPAD, 16)[:N]
    y = x
    for li, pre in enumerate(prefixes):
        if li > 0:
            prev = prefixes[li - 1]
            y = _bnleaky(y, params[prev + '_gamma'], params[prev + '_beta'])
        if has_ew:
            xl, xr, ns, dsf = _prep(y, params[pre + '_Wl'], params[pre + '_Wr'],
                                    params[pre + '_att'], lea, params[pre + '_We'])
            ns_p, ds_p = _pad_self(ns, dsf)
            eaw = _eaw(ea_pad, params[pre + '_We'])
            y = edge_call(src_p, dst_p, xl, xr, eaw,
                          params[pre + '_att'], params[pre + '_b'],
                          ns_p, ds_p)[:N]
        else:
            xl, xr, ns, dsf = _prep(y, params[pre + '_Wl'], params[pre + '_Wr'],
                                    params[pre + '_att'])
            ns_p, ds_p = _pad_self(ns, dsf)
            y = edge_call(src_p, dst_p, xl, xr,
                          params[pre + '_att'], params[pre + '_b'],
                          ns_p, ds_p)[:N]
    last = prefixes[-1]
    y = _bnleaky(y, params[last + '_gamma'], params[last + '_beta'])
    return _pool(y, batch)


def kernel(global_data, segment_x, segment_edge_index, segment_edge_attr,
           segment_batch, dense_x, dense_edge_index, dense_batch, params):
    p = params
    g = pl.pallas_call(
        _global_kernel,
        out_shape=jax.ShapeDtypeStruct((32, F), FP32),
    )(global_data, p['g_W'], p['g_b'], p['g_gamma'], p['g_beta'])

    s_pool = _gat_path(segment_x, segment_edge_index[0], segment_edge_index[1],
                       160768, p, ('s0', 's1'), segment_batch,
                       ea=segment_edge_attr)
    d_pool = _gat_path(dense_x, dense_edge_index[0], dense_edge_index[1],
                       321536, p, ('d0', 'd1'), dense_batch)

    return pl.pallas_call(
        _head_kernel,
        out_shape=jax.ShapeDtypeStruct((32, 2), FP32),
    )(g, s_pool, d_pool, p['o_W'], p['o_b'])
